# baseline (speedup 1.0000x reference)
import functools

import jax
import jax.numpy as jnp
from jax import lax
from jax.experimental import pallas as pl
from jax.experimental.pallas import tpu as pltpu
from jax.experimental.pallas import tpu_sc as plsc

BATCH = 4096
N_LOG = 64 * 64
N_PP = 8 * 24
P = 1858
L = 16
NSLOT = (P + L - 1) // L
LAST_OFF = P - L
R = 8


def _sc_policy_gather(logits1d, pp1d, idx):
    info = plsc.get_sparse_core_info()
    nc, ns = info.num_cores, info.num_subcores
    nw = nc * ns
    rows_per_w = BATCH // nw
    n_chunks = rows_per_w // R

    mesh = plsc.VectorSubcoreMesh(core_axis_name="c", subcore_axis_name="s")

    @functools.partial(
        pl.kernel,
        mesh=mesh,
        out_type=jax.ShapeDtypeStruct((BATCH * P,), jnp.float32),
        compiler_params=pltpu.CompilerParams(needs_layout_passes=False),
        scratch_types=[
            pltpu.VMEM((P,), jnp.int32),
            pltpu.VMEM((NSLOT * L,), jnp.int32),
            pltpu.VMEM((NSLOT * L,), jnp.int32),
            pltpu.VMEM((NSLOT * L,), jnp.int32),
            pltpu.VMEM((R * N_LOG,), jnp.float32),
            pltpu.VMEM((R * N_PP,), jnp.float32),
            pltpu.VMEM((R * P,), jnp.float32),
        ],
    )
    def k(log_hbm, pp_hbm, idx_hbm, out_hbm,
          idx_v, iraw_v, ilog_v, ipp_v, log_b, pp_b, out_b):
        wid = lax.axis_index("s") * nc + lax.axis_index("c")
        base = wid * rows_per_w

        pltpu.sync_copy(idx_hbm, idx_v)

        def prep(kslot, carry):
            off = jnp.where(kslot == NSLOT - 1, LAST_OFF, kslot * L)
            iv = idx_v[pl.ds(off, L)]
            iraw_v[pl.ds(kslot * L, L)] = iv
            ilog_v[pl.ds(kslot * L, L)] = jnp.minimum(iv, N_LOG - 1)
            ipp_v[pl.ds(kslot * L, L)] = jnp.clip(iv - N_LOG, 0, N_PP - 1)
            return carry

        lax.fori_loop(0, NSLOT, prep, 0)

        def chunk_body(c, carry):
            r0 = base + c * R
            pltpu.sync_copy(log_hbm.at[pl.ds(r0 * N_LOG, R * N_LOG)], log_b)
            pltpu.sync_copy(pp_hbm.at[pl.ds(r0 * N_PP, R * N_PP)], pp_b)

            def slot_body(kslot, inner):
                iv = iraw_v[pl.ds(kslot * L, L)]
                il = ilog_v[pl.ds(kslot * L, L)]
                ip = ipp_v[pl.ds(kslot * L, L)]
                msk = iv < N_LOG
                off = jnp.where(kslot == NSLOT - 1, LAST_OFF, kslot * L)
                for r in range(R):
                    vlog = plsc.load_gather(
                        log_b, [il + jnp.full((L,), r * N_LOG, jnp.int32)])
                    vpp = plsc.load_gather(
                        pp_b, [ip + jnp.full((L,), r * N_PP, jnp.int32)])
                    out_b[pl.ds(off + r * P, L)] = jnp.where(msk, vlog, vpp)
                return inner

            lax.fori_loop(0, NSLOT, slot_body, 0)
            pltpu.sync_copy(out_b, out_hbm.at[pl.ds(r0 * P, R * P)])
            return carry

        lax.fori_loop(0, n_chunks, chunk_body, 0)

    return k(logits1d, pp1d, idx)


def kernel(logits, pp_logits, idx):
    out_flat = _sc_policy_gather(
        logits.reshape(BATCH * N_LOG),
        pp_logits.reshape(BATCH * N_PP),
        idx,
    )
    return out_flat.reshape(BATCH, P)


# speedup vs baseline: 1.2077x; 1.2077x over previous
import functools

import jax
import jax.numpy as jnp
from jax import lax
from jax.experimental import pallas as pl
from jax.experimental.pallas import tpu as pltpu
from jax.experimental.pallas import tpu_sc as plsc

BATCH = 4096
N_LOG = 64 * 64
N_PP = 8 * 24
FLAT = N_LOG + N_PP
P = 1858
L = 16
NSLOT = (P + L - 1) // L
LAST_OFF = P - L
R = 8


def _sc_policy_gather(logits1d, pp1d, idx):
    info = plsc.get_sparse_core_info()
    nc, ns = info.num_cores, info.num_subcores
    nw = nc * ns
    rows_per_w = BATCH // nw
    n_chunks = rows_per_w // R

    mesh = plsc.VectorSubcoreMesh(core_axis_name="c", subcore_axis_name="s")

    @functools.partial(
        pl.kernel,
        mesh=mesh,
        out_type=jax.ShapeDtypeStruct((BATCH * P,), jnp.float32),
        compiler_params=pltpu.CompilerParams(needs_layout_passes=False),
        scratch_types=[
            pltpu.VMEM((P,), jnp.int32),
            pltpu.VMEM((R * FLAT,), jnp.float32),
            pltpu.VMEM((R * FLAT,), jnp.float32),
            pltpu.VMEM((R * P,), jnp.float32),
            pltpu.VMEM((R * P,), jnp.float32),
            pltpu.SemaphoreType.DMA,
            pltpu.SemaphoreType.DMA,
            pltpu.SemaphoreType.DMA,
            pltpu.SemaphoreType.DMA,
        ],
    )
    def k(log_hbm, pp_hbm, idx_hbm, out_hbm,
          idx_v, comb_a, comb_b, out_a, out_b,
          sin_a, sin_b, sout_a, sout_b):
        wid = lax.axis_index("s") * nc + lax.axis_index("c")
        base = wid * rows_per_w

        pltpu.sync_copy(idx_hbm, idx_v)

        def issue_in(c, comb, sem):
            r0 = base + c * R
            for r in range(R):
                pltpu.async_copy(
                    log_hbm.at[pl.ds((r0 + r) * N_LOG, N_LOG)],
                    comb.at[pl.ds(r * FLAT, N_LOG)], sem)
                pltpu.async_copy(
                    pp_hbm.at[pl.ds((r0 + r) * N_PP, N_PP)],
                    comb.at[pl.ds(r * FLAT + N_LOG, N_PP)], sem)

        def wait_in(comb, sem):
            pltpu.make_async_copy(
                log_hbm.at[pl.ds(0, R * FLAT)], comb, sem).wait()

        def issue_out(c, out_v, sem):
            r0 = base + c * R
            pltpu.async_copy(out_v, out_hbm.at[pl.ds(r0 * P, R * P)], sem)

        def wait_out(out_v, sem):
            pltpu.make_async_copy(
                out_v, out_hbm.at[pl.ds(0, R * P)], sem).wait()

        def compute(comb, out_v):
            def slot(kk, carry):
                off = jnp.where(kk == NSLOT - 1, LAST_OFF, kk * L)
                iv = idx_v[pl.ds(off, L)]
                for r in range(R):
                    v = plsc.load_gather(
                        comb, [iv + jnp.full((L,), r * FLAT, jnp.int32)])
                    out_v[pl.ds(off + r * P, L)] = v
                return carry

            lax.fori_loop(0, NSLOT, slot, 0)

        issue_in(0, comb_a, sin_a)
        issue_in(1, comb_b, sin_b)

        def pair(g, carry):
            c0 = 2 * g
            wait_in(comb_a, sin_a)

            @pl.when(g > 0)
            def _():
                wait_out(out_a, sout_a)

            compute(comb_a, out_a)
            issue_out(c0, out_a, sout_a)

            @pl.when(g < n_chunks // 2 - 1)
            def _():
                issue_in(c0 + 2, comb_a, sin_a)

            wait_in(comb_b, sin_b)

            @pl.when(g > 0)
            def _():
                wait_out(out_b, sout_b)

            compute(comb_b, out_b)
            issue_out(c0 + 1, out_b, sout_b)

            @pl.when(g < n_chunks // 2 - 1)
            def _():
                issue_in(c0 + 3, comb_b, sin_b)

            return carry

        lax.fori_loop(0, n_chunks // 2, pair, 0)
        wait_out(out_a, sout_a)
        wait_out(out_b, sout_b)

    return k(logits1d, pp1d, idx)


def kernel(logits, pp_logits, idx):
    out_flat = _sc_policy_gather(
        logits.reshape(BATCH * N_LOG),
        pp_logits.reshape(BATCH * N_PP),
        idx,
    )
    return out_flat.reshape(BATCH, P)
